# baseline (speedup 1.0000x reference)
import functools

import jax
import jax.numpy as jnp
from jax import lax
from jax.experimental import pallas as pl
from jax.experimental.pallas import tpu as pltpu
from jax.experimental.pallas import tpu_sc as plsc

BATCH = 4096
SEQ = 200
DIM = 64
NSPLIT = 4
SUBD = DIM // NSPLIT
NUM_WORKERS = 32
B_PER_W = BATCH // NUM_WORKERS
LANES = 16


def _bow_body(idx_hbm, t0, t1, t2, t3, out_hbm, idx_v,
              b00, b01, b10, b11, b20, b21, b30, b31, out_v,
              s00, s01, s10, s11, s20, s21, s30, s31):
    wid = lax.axis_index("s") * 2 + lax.axis_index("c")
    tables = (t0, t1, t2, t3)
    bufs = ((b00, b01), (b10, b11), (b20, b21), (b30, b31))
    sems = ((s00, s01), (s10, s11), (s20, s21), (s30, s31))

    pltpu.sync_copy(idx_hbm.at[pl.ds(wid * B_PER_W * SEQ, B_PER_W * SEQ)],
                    idx_v)

    def gather_row(bb, p):
        for t in range(NSPLIT):
            pltpu.async_copy(
                tables[t].at[idx_v.at[pl.ds(bb * SEQ, SEQ)]],
                bufs[t][p], sems[t][p])

    gather_row(0, 0)
    gather_row(1, 1)

    def accumulate(p):
        def r_body(r, a):
            return tuple(x + bufs[t][p][r, pl.ds(0, LANES)]
                         for t, x in enumerate(a))
        return lax.fori_loop(
            0, SEQ, r_body,
            tuple(jnp.zeros((LANES,), jnp.float32) for _ in range(NSPLIT)))

    def b_body(b, carry):
        for p in (0, 1):
            bb = 2 * b + p
            for t in range(NSPLIT):
                pltpu.make_async_copy(
                    tables[t].at[idx_v.at[pl.ds(0, SEQ)]],
                    bufs[t][p], sems[t][p]).wait()
            accs = accumulate(p)
            for t in range(NSPLIT):
                out_v[bb, pl.ds(t * SUBD, SUBD)] = accs[t]
            nxt = bb + 2

            @pl.when(nxt < B_PER_W)
            def _():
                gather_row(nxt, p)

        return carry

    lax.fori_loop(0, B_PER_W // 2, b_body, 0)

    pltpu.sync_copy(out_v, out_hbm.at[pl.ds(wid * B_PER_W, B_PER_W)])


@functools.partial(
    pl.kernel,
    mesh=plsc.VectorSubcoreMesh(core_axis_name="c", subcore_axis_name="s"),
    out_type=jax.ShapeDtypeStruct((BATCH, DIM), jnp.float32),
    scratch_types=(
        [pltpu.VMEM((B_PER_W * SEQ,), jnp.int32)]
        + [pltpu.VMEM((SEQ, SUBD), jnp.float32) for _ in range(2 * NSPLIT)]
        + [pltpu.VMEM((B_PER_W, DIM), jnp.float32)]
        + [pltpu.SemaphoreType.DMA for _ in range(2 * NSPLIT)]
    ),
    compiler_params=pltpu.CompilerParams(use_tc_tiling_on_sc=False),
)
def _bow_sc(idx_hbm, t0, t1, t2, t3, out_hbm, idx_v,
            b00, b01, b10, b11, b20, b21, b30, b31, out_v,
            s00, s01, s10, s11, s20, s21, s30, s31):
    _bow_body(idx_hbm, t0, t1, t2, t3, out_hbm, idx_v,
              b00, b01, b10, b11, b20, b21, b30, b31, out_v,
              s00, s01, s10, s11, s20, s21, s30, s31)


@jax.jit
def kernel(indices, table):
    idx = indices.astype(jnp.int32).reshape(-1)
    slices = [table[:, t * SUBD:(t + 1) * SUBD] for t in range(NSPLIT)]
    return _bow_sc(idx, *slices)


# speedup vs baseline: 3.5138x; 3.5138x over previous
import functools

import jax
import jax.numpy as jnp
from jax import lax
from jax.experimental import pallas as pl
from jax.experimental.pallas import tpu as pltpu
from jax.experimental.pallas import tpu_sc as plsc

BATCH = 4096
SEQ = 200
DIM = 64
NUM_WORKERS = 32
B_PER_W = BATCH // NUM_WORKERS
LANES = 16
VECS = DIM // LANES


def _bow_body(idx_hbm, table_hbm, out_hbm, idx_v, rows0, rows1, out_v,
              sem0, sem1):
    wid = lax.axis_index("s") * 2 + lax.axis_index("c")

    pltpu.sync_copy(idx_hbm.at[pl.ds(wid * B_PER_W * SEQ, B_PER_W * SEQ)],
                    idx_v)

    pltpu.async_copy(table_hbm.at[idx_v.at[pl.ds(0, SEQ)]], rows0, sem0)
    pltpu.async_copy(table_hbm.at[idx_v.at[pl.ds(SEQ, SEQ)]], rows1, sem1)

    def accumulate(buf, accs):
        def r_body(r, a):
            a = [x + buf[2 * r, pl.ds(d * LANES, LANES)]
                 for d, x in enumerate(a)]
            return tuple(x + buf[2 * r + 1, pl.ds(d * LANES, LANES)]
                         for d, x in enumerate(a))
        return lax.fori_loop(0, SEQ // 2, r_body, accs)

    def b_body(b, carry):
        for p, (buf, sem) in enumerate(((rows0, sem0), (rows1, sem1))):
            bb = 2 * b + p
            pltpu.make_async_copy(table_hbm.at[idx_v.at[pl.ds(0, SEQ)]],
                                  buf, sem).wait()
            accs = tuple(jnp.zeros((LANES,), jnp.float32)
                         for _ in range(VECS))
            accs = accumulate(buf, accs)
            for d in range(VECS):
                out_v[bb, pl.ds(d * LANES, LANES)] = accs[d]
            nxt = bb + 2

            @pl.when(nxt < B_PER_W)
            def _():
                pltpu.async_copy(table_hbm.at[idx_v.at[pl.ds(nxt * SEQ, SEQ)]],
                                 buf, sem)

        return carry

    lax.fori_loop(0, B_PER_W // 2, b_body, 0)

    pltpu.sync_copy(out_v, out_hbm.at[pl.ds(wid * B_PER_W, B_PER_W)])


@functools.partial(
    pl.kernel,
    mesh=plsc.VectorSubcoreMesh(core_axis_name="c", subcore_axis_name="s"),
    out_type=jax.ShapeDtypeStruct((BATCH, DIM), jnp.float32),
    scratch_types=[
        pltpu.VMEM((B_PER_W * SEQ,), jnp.int32),
        pltpu.VMEM((SEQ, DIM), jnp.float32),
        pltpu.VMEM((SEQ, DIM), jnp.float32),
        pltpu.VMEM((B_PER_W, DIM), jnp.float32),
        pltpu.SemaphoreType.DMA,
        pltpu.SemaphoreType.DMA,
    ],
    compiler_params=pltpu.CompilerParams(use_tc_tiling_on_sc=False),
)
def _bow_sc(idx_hbm, table_hbm, out_hbm, idx_v, rows0, rows1, out_v,
            sem0, sem1):
    _bow_body(idx_hbm, table_hbm, out_hbm, idx_v, rows0, rows1, out_v,
              sem0, sem1)


@jax.jit
def kernel(indices, table):
    return _bow_sc(indices.astype(jnp.int32).reshape(-1), table)


# speedup vs baseline: 3.7315x; 1.0620x over previous
import functools

import jax
import jax.numpy as jnp
from jax import lax
from jax.experimental import pallas as pl
from jax.experimental.pallas import tpu as pltpu
from jax.experimental.pallas import tpu_sc as plsc

BATCH = 4096
SEQ = 200
DIM = 64
NUM_WORKERS = 32
B_PER_W = BATCH // NUM_WORKERS
LANES = 16
VECS = DIM // LANES


def _bow_body(idx_hbm, table_hbm, out_hbm, idx_v, rows0, rows1, rows2, rows3,
              out_v, sem0, sem1, sem2, sem3):
    wid = lax.axis_index("s") * 2 + lax.axis_index("c")

    pltpu.sync_copy(idx_hbm.at[pl.ds(wid * B_PER_W * SEQ, B_PER_W * SEQ)],
                    idx_v)

    bufs = (rows0, rows1, rows2, rows3)
    sems = (sem0, sem1, sem2, sem3)
    for p in range(4):
        pltpu.async_copy(table_hbm.at[idx_v.at[pl.ds(p * SEQ, SEQ)]],
                         bufs[p], sems[p])

    def accumulate(buf, accs):
        def r_body(r, a):
            a = [x + buf[2 * r, pl.ds(d * LANES, LANES)]
                 for d, x in enumerate(a)]
            return tuple(x + buf[2 * r + 1, pl.ds(d * LANES, LANES)]
                         for d, x in enumerate(a))
        return lax.fori_loop(0, SEQ // 2, r_body, accs)

    def b_body(b, carry):
        for p in range(4):
            buf, sem = bufs[p], sems[p]
            bb = 4 * b + p
            pltpu.make_async_copy(table_hbm.at[idx_v.at[pl.ds(0, SEQ)]],
                                  buf, sem).wait()
            accs = tuple(jnp.zeros((LANES,), jnp.float32)
                         for _ in range(VECS))
            accs = accumulate(buf, accs)
            for d in range(VECS):
                out_v[bb, pl.ds(d * LANES, LANES)] = accs[d]
            nxt = bb + 4

            @pl.when(nxt < B_PER_W)
            def _():
                pltpu.async_copy(table_hbm.at[idx_v.at[pl.ds(nxt * SEQ, SEQ)]],
                                 buf, sem)

        return carry

    lax.fori_loop(0, B_PER_W // 4, b_body, 0)

    pltpu.sync_copy(out_v, out_hbm.at[pl.ds(wid * B_PER_W, B_PER_W)])


@functools.partial(
    pl.kernel,
    mesh=plsc.VectorSubcoreMesh(core_axis_name="c", subcore_axis_name="s"),
    out_type=jax.ShapeDtypeStruct((BATCH, DIM), jnp.float32),
    scratch_types=[
        pltpu.VMEM((B_PER_W * SEQ,), jnp.int32),
        pltpu.VMEM((SEQ, DIM), jnp.float32),
        pltpu.VMEM((SEQ, DIM), jnp.float32),
        pltpu.VMEM((SEQ, DIM), jnp.float32),
        pltpu.VMEM((SEQ, DIM), jnp.float32),
        pltpu.VMEM((B_PER_W, DIM), jnp.float32),
        pltpu.SemaphoreType.DMA,
        pltpu.SemaphoreType.DMA,
        pltpu.SemaphoreType.DMA,
        pltpu.SemaphoreType.DMA,
    ],
    compiler_params=pltpu.CompilerParams(use_tc_tiling_on_sc=False),
)
def _bow_sc(idx_hbm, table_hbm, out_hbm, idx_v, rows0, rows1, rows2, rows3,
            out_v, sem0, sem1, sem2, sem3):
    _bow_body(idx_hbm, table_hbm, out_hbm, idx_v, rows0, rows1, rows2, rows3,
              out_v, sem0, sem1, sem2, sem3)


@jax.jit
def kernel(indices, table):
    return _bow_sc(indices.astype(jnp.int32).reshape(-1), table)
